# baseline (speedup 1.0000x reference)
import functools

import jax
import jax.numpy as jnp
from jax import lax
from jax.experimental import pallas as pl
from jax.experimental.pallas import tpu as pltpu
from jax.experimental.pallas import tpu_sc as plsc

_NC, _NS, _L = 2, 16, 16
_NW = _NC * _NS


@functools.partial(jax.jit, static_argnums=(3, 4, 5))
def _sc_symmetry_loss(sp_flat, cp_flat, params, N, G3, G):
    B = sp_flat.size // (3 * N)
    CH = 8192
    NCHUNK = N // CH

    mesh = plsc.VectorSubcoreMesh(
        core_axis_name="c", subcore_axis_name="s",
        num_cores=_NC, num_subcores=_NS)

    @functools.partial(
        pl.kernel,
        out_type=(jax.ShapeDtypeStruct((_NW * _L,), jnp.float32),
                  jax.ShapeDtypeStruct((_NW * _L,), jnp.float32)),
        mesh=mesh,
        compiler_params=pltpu.CompilerParams(needs_layout_passes=False),
        scratch_types=[
            pltpu.VMEM((3 * G3,), jnp.float32),
            pltpu.VMEM((3 * CH,), jnp.float32),
            pltpu.VMEM((13 * _L,), jnp.float32),
            pltpu.VMEM((_L,), jnp.float32),
            pltpu.VMEM((_L,), jnp.float32),
        ],
    )
    def launch(sp_ref, cp_ref, par_ref, oref_ref, orot_ref,
               tab, pbuf, pvm, obuf_r, obuf_o):
        wid = lax.axis_index("s") * _NC + lax.axis_index("c")
        b = wid // 8

        for c in range(3):
            pltpu.sync_copy(cp_ref.at[pl.ds((c * B + b) * G3, G3)],
                            tab.at[pl.ds(c * G3, G3)])
        pltpu.sync_copy(par_ref.at[pl.ds(wid * (13 * _L), 13 * _L)], pvm)

        def bc(i):
            return pvm[pl.ds(i * _L, _L)]

        n0, n1, n2 = bc(0), bc(1), bc(2)
        w0, w1, w2 = bc(3), bc(4), bc(5)
        e = bc(6)
        s0, s1, s2 = bc(7), bc(8), bc(9)
        bnd, gsf, gmax = bc(10), bc(11), bc(12)
        lane = lax.broadcasted_iota(jnp.int32, (_L,), 0)

        def vox(t):
            v = (t + bnd) * gsf
            v = jnp.maximum(v, 0.0)
            v = jnp.minimum(v, gmax)
            return v.astype(jnp.int32)

        def body(j, acc_r, acc_o):
            base = lane + j * _L
            px = plsc.load_gather(pbuf, [base])
            py = plsc.load_gather(pbuf, [base + CH])
            pz = plsc.load_gather(pbuf, [base + 2 * CH])

            dst = px * w0 + py * w1 + pz * w2 + e
            tx = px - dst * n0
            ty = py - dst * n1
            tz = pz - dst * n2
            f = (vox(tx) * G + vox(ty)) * G + vox(tz)
            cx = plsc.load_gather(tab, [f])
            cy = plsc.load_gather(tab, [f + G3])
            cz = plsc.load_gather(tab, [f + 2 * G3])
            dx, dy, dz = tx - cx, ty - cy, tz - cz
            acc_r = acc_r + ((dx * dx + dy * dy) + dz * dz)

            ux = s0 * px
            uy = s1 * py
            uz = s2 * pz
            g = (vox(ux) * G + vox(uy)) * G + vox(uz)
            qx = plsc.load_gather(tab, [g])
            qy = plsc.load_gather(tab, [g + G3])
            qz = plsc.load_gather(tab, [g + 2 * G3])
            ex, ey, ez = ux - qx, uy - qy, uz - qz
            acc_o = acc_o + ((ex * ex + ey * ey) + ez * ez)
            return acc_r, acc_o

        UN = 4
        zero = jnp.zeros((_L,), jnp.float32)
        accs = (zero,) * (2 * UN)
        for ch in range(NCHUNK):
            for c in range(3):
                pltpu.sync_copy(
                    sp_ref.at[pl.ds((c * B + b) * N + ch * CH, CH)],
                    pbuf.at[pl.ds(c * CH, CH)])

            def step(i, accs):
                out = []
                for k in range(UN):
                    out.extend(body(i + k, accs[2 * k], accs[2 * k + 1]))
                return tuple(out)

            accs = plsc.parallel_loop(
                0, CH // _L, step=UN, carry=accs)(step)

        acc_r = (accs[0] + accs[2]) + (accs[4] + accs[6])
        acc_o = (accs[1] + accs[3]) + (accs[5] + accs[7])
        obuf_r[...] = acc_r
        obuf_o[...] = acc_o
        pltpu.sync_copy(obuf_r, oref_ref.at[pl.ds(wid * _L, _L)])
        pltpu.sync_copy(obuf_o, orot_ref.at[pl.ds(wid * _L, _L)])

    return launch(sp_flat, cp_flat, params)


def kernel(sample_points, closest_points, bound, grid_size, planes, axes):
    B, N, _ = sample_points.shape
    G3 = closest_points.shape[1]
    G = round(G3 ** (1.0 / 3.0))
    T = planes.shape[0]

    n = planes[:, :, :3]
    d = planes[:, :, 3]
    s = jnp.sum(n * n, axis=2) + 1e-12
    inv = 2.0 / s
    w = inv[:, :, None] * n
    e = inv * d
    srot = -(axes[:, :, 1:] ** 2)

    bnd = bound[0].astype(jnp.float32)
    gsf = jnp.asarray(grid_size, jnp.float32)
    scal = jnp.stack([bnd, gsf, gsf - 1.0])

    def tb(x):
        return jnp.transpose(x, (1, 0, 2)).reshape(B * T, -1)

    params = jnp.concatenate([
        tb(n), tb(w), tb(e[:, :, None]), tb(srot),
        jnp.broadcast_to(scal, (B * T, 3)),
    ], axis=1).astype(jnp.float32)
    params = jnp.broadcast_to(params[:, :, None], (B * T, 13, 16))

    spT = jnp.transpose(sample_points, (2, 0, 1)).reshape(-1)
    cpT = jnp.transpose(closest_points, (2, 0, 1)).reshape(-1)

    oref, orot = _sc_symmetry_loss(spT, cpT, params.reshape(-1), N, G3, G)
    denom = jnp.float32(B * 3)
    return ((jnp.sum(oref) / denom).reshape(1),
            (jnp.sum(orot) / denom).reshape(1))


# speedup vs baseline: 1.0559x; 1.0559x over previous
import functools

import jax
import jax.numpy as jnp
from jax import lax
from jax.experimental import pallas as pl
from jax.experimental.pallas import tpu as pltpu
from jax.experimental.pallas import tpu_sc as plsc

_NC, _NS, _L = 2, 16, 16
_NW = _NC * _NS


@functools.partial(jax.jit, static_argnums=(3, 4, 5))
def _sc_symmetry_loss(sp_flat, cp_flat, params, N, G3, G):
    B = sp_flat.size // (3 * N)
    CH = 4096
    NCHUNK = N // CH

    mesh = plsc.VectorSubcoreMesh(
        core_axis_name="c", subcore_axis_name="s",
        num_cores=_NC, num_subcores=_NS)

    @functools.partial(
        pl.kernel,
        out_type=(jax.ShapeDtypeStruct((_NW * _L,), jnp.float32),
                  jax.ShapeDtypeStruct((_NW * _L,), jnp.float32)),
        mesh=mesh,
        compiler_params=pltpu.CompilerParams(needs_layout_passes=False),
        scratch_types=[
            pltpu.VMEM((3 * G3,), jnp.float32),
            pltpu.VMEM((2 * 3 * CH,), jnp.float32),
            pltpu.VMEM((13 * _L,), jnp.float32),
            pltpu.VMEM((_L,), jnp.float32),
            pltpu.VMEM((_L,), jnp.float32),
            pltpu.SemaphoreType.DMA,
            pltpu.SemaphoreType.DMA,
            pltpu.SemaphoreType.DMA,
        ],
    )
    def launch(sp_ref, cp_ref, par_ref, oref_ref, orot_ref,
               tab, pbuf, pvm, obuf_r, obuf_o, sem_t, sem_c0, sem_c1):
        wid = lax.axis_index("s") * _NC + lax.axis_index("c")
        b = wid // 8
        csem = (sem_c0, sem_c1)

        def start_chunk(ch):
            buf = ch % 2
            cps = []
            for c in range(3):
                cps.append(pltpu.async_copy(
                    sp_ref.at[pl.ds((c * B + b) * N + ch * CH, CH)],
                    pbuf.at[pl.ds((3 * buf + c) * CH, CH)], csem[buf]))
            return cps

        tcps = [pltpu.async_copy(cp_ref.at[pl.ds((c * B + b) * G3, G3)],
                                 tab.at[pl.ds(c * G3, G3)], sem_t)
                for c in range(3)]
        tcps.append(pltpu.async_copy(
            par_ref.at[pl.ds(wid * (13 * _L), 13 * _L)], pvm, sem_t))
        chunk_cps = start_chunk(0)
        for cp_ in tcps:
            cp_.wait()

        def bc(i):
            return pvm[pl.ds(i * _L, _L)]

        n0, n1, n2 = bc(0), bc(1), bc(2)
        w0, w1, w2 = bc(3), bc(4), bc(5)
        e = bc(6)
        s0, s1, s2 = bc(7), bc(8), bc(9)
        bnd, gsf, gmax = bc(10), bc(11), bc(12)
        lane = lax.broadcasted_iota(jnp.int32, (_L,), 0)

        def vox(t):
            v = (t + bnd) * gsf
            v = jnp.maximum(v, 0.0)
            v = jnp.minimum(v, gmax)
            return v.astype(jnp.int32)

        def body(bufref, j, acc_r, acc_o):
            base = lane + j * _L
            px = plsc.load_gather(bufref, [base])
            py = plsc.load_gather(bufref, [base + CH])
            pz = plsc.load_gather(bufref, [base + 2 * CH])

            dst = px * w0 + py * w1 + pz * w2 + e
            tx = px - dst * n0
            ty = py - dst * n1
            tz = pz - dst * n2
            f = (vox(tx) * G + vox(ty)) * G + vox(tz)
            cx = plsc.load_gather(tab, [f])
            cy = plsc.load_gather(tab, [f + G3])
            cz = plsc.load_gather(tab, [f + 2 * G3])
            dx, dy, dz = tx - cx, ty - cy, tz - cz
            acc_r = acc_r + ((dx * dx + dy * dy) + dz * dz)

            ux = s0 * px
            uy = s1 * py
            uz = s2 * pz
            g = (vox(ux) * G + vox(uy)) * G + vox(uz)
            qx = plsc.load_gather(tab, [g])
            qy = plsc.load_gather(tab, [g + G3])
            qz = plsc.load_gather(tab, [g + 2 * G3])
            ex, ey, ez = ux - qx, uy - qy, uz - qz
            acc_o = acc_o + ((ex * ex + ey * ey) + ez * ez)
            return acc_r, acc_o

        UN = 4
        zero = jnp.zeros((_L,), jnp.float32)
        accs = (zero,) * (2 * UN)
        for ch in range(NCHUNK):
            for cp_ in chunk_cps:
                cp_.wait()
            if ch + 1 < NCHUNK:
                next_cps = start_chunk(ch + 1)
            else:
                next_cps = []
            bufref = pbuf.at[pl.ds((ch % 2) * 3 * CH, 3 * CH)]

            def step(i, accs, bufref=bufref):
                out = []
                for k in range(UN):
                    out.extend(
                        body(bufref, i + k, accs[2 * k], accs[2 * k + 1]))
                return tuple(out)

            accs = plsc.parallel_loop(
                0, CH // _L, step=UN, carry=accs)(step)
            chunk_cps = next_cps

        acc_r = (accs[0] + accs[2]) + (accs[4] + accs[6])
        acc_o = (accs[1] + accs[3]) + (accs[5] + accs[7])
        obuf_r[...] = acc_r
        obuf_o[...] = acc_o
        pltpu.sync_copy(obuf_r, oref_ref.at[pl.ds(wid * _L, _L)])
        pltpu.sync_copy(obuf_o, orot_ref.at[pl.ds(wid * _L, _L)])

    return launch(sp_flat, cp_flat, params)


def kernel(sample_points, closest_points, bound, grid_size, planes, axes):
    B, N, _ = sample_points.shape
    G3 = closest_points.shape[1]
    G = round(G3 ** (1.0 / 3.0))
    T = planes.shape[0]

    n = planes[:, :, :3]
    d = planes[:, :, 3]
    s = jnp.sum(n * n, axis=2) + 1e-12
    inv = 2.0 / s
    w = inv[:, :, None] * n
    e = inv * d
    srot = -(axes[:, :, 1:] ** 2)

    bnd = bound[0].astype(jnp.float32)
    gsf = jnp.asarray(grid_size, jnp.float32)
    scal = jnp.stack([bnd, gsf, gsf - 1.0])

    def tb(x):
        return jnp.transpose(x, (1, 0, 2)).reshape(B * T, -1)

    params = jnp.concatenate([
        tb(n), tb(w), tb(e[:, :, None]), tb(srot),
        jnp.broadcast_to(scal, (B * T, 3)),
    ], axis=1).astype(jnp.float32)
    params = jnp.broadcast_to(params[:, :, None], (B * T, 13, 16))

    spT = jnp.transpose(sample_points, (2, 0, 1)).reshape(-1)
    cpT = jnp.transpose(closest_points, (2, 0, 1)).reshape(-1)

    oref, orot = _sc_symmetry_loss(spT, cpT, params.reshape(-1), N, G3, G)
    denom = jnp.float32(B * 3)
    return ((jnp.sum(oref) / denom).reshape(1),
            (jnp.sum(orot) / denom).reshape(1))


# speedup vs baseline: 1.0660x; 1.0095x over previous
import functools

import jax
import jax.numpy as jnp
from jax import lax
from jax.experimental import pallas as pl
from jax.experimental.pallas import tpu as pltpu
from jax.experimental.pallas import tpu_sc as plsc

_NC, _NS, _L = 2, 16, 16
_NW = _NC * _NS


@functools.partial(jax.jit, static_argnums=(3, 4, 5))
def _sc_symmetry_loss(sp_flat, cp_flat, params, N, G3, G):
    B = sp_flat.size // (3 * N)
    CH = 4096
    NCHUNK = N // CH

    mesh = plsc.VectorSubcoreMesh(
        core_axis_name="c", subcore_axis_name="s",
        num_cores=_NC, num_subcores=_NS)

    @functools.partial(
        pl.kernel,
        out_type=(jax.ShapeDtypeStruct((_NW * _L,), jnp.float32),
                  jax.ShapeDtypeStruct((_NW * _L,), jnp.float32)),
        mesh=mesh,
        compiler_params=pltpu.CompilerParams(needs_layout_passes=False),
        scratch_types=[
            pltpu.VMEM((3 * G3,), jnp.float32),
            pltpu.VMEM((2 * 3 * CH,), jnp.float32),
            pltpu.VMEM((24,), jnp.float32),
            pltpu.VMEM((_L,), jnp.float32),
            pltpu.VMEM((_L,), jnp.float32),
            pltpu.SemaphoreType.DMA,
            pltpu.SemaphoreType.DMA,
            pltpu.SemaphoreType.DMA,
        ],
    )
    def launch(sp_ref, cp_ref, par_ref, oref_ref, orot_ref,
               tab, pbuf, pvm, obuf_r, obuf_o, sem_t, sem_c0, sem_c1):
        wid = lax.axis_index("s") * _NC + lax.axis_index("c")
        b = wid // 8
        csem = (sem_c0, sem_c1)

        def start_chunk(ch):
            buf = ch % 2
            cps = []
            for c in range(3):
                cps.append(pltpu.async_copy(
                    sp_ref.at[pl.ds((c * B + b) * N + ch * CH, CH)],
                    pbuf.at[pl.ds((3 * buf + c) * CH, CH)], csem[buf]))
            return cps

        tcps = [pltpu.async_copy(cp_ref.at[pl.ds((c * B + b) * G3, G3)],
                                 tab.at[pl.ds(c * G3, G3)], sem_t)
                for c in range(3)]
        tcps.append(pltpu.async_copy(
            par_ref.at[pl.ds(wid * 16, 16)], pvm.at[pl.ds(8, 16)], sem_t))
        chunk_cps = start_chunk(0)
        for cp_ in tcps:
            cp_.wait()

        def bc(i):
            return plsc.load_gather(pvm, [jnp.full((_L,), 8 + i, jnp.int32)])

        n0, n1, n2 = bc(0), bc(1), bc(2)
        w0, w1, w2 = bc(3), bc(4), bc(5)
        e = bc(6)
        s0, s1, s2 = bc(7), bc(8), bc(9)
        bnd, gsf, gmax = bc(10), bc(11), bc(12)
        lane = lax.broadcasted_iota(jnp.int32, (_L,), 0)

        def vox(t):
            v = (t + bnd) * gsf
            v = jnp.maximum(v, 0.0)
            v = jnp.minimum(v, gmax)
            return v.astype(jnp.int32)

        def body(bufref, j, acc_r, acc_o):
            base = lane + j * _L
            px = plsc.load_gather(bufref, [base])
            py = plsc.load_gather(bufref, [base + CH])
            pz = plsc.load_gather(bufref, [base + 2 * CH])

            dst = px * w0 + py * w1 + pz * w2 + e
            tx = px - dst * n0
            ty = py - dst * n1
            tz = pz - dst * n2
            f = (vox(tx) * G + vox(ty)) * G + vox(tz)
            cx = plsc.load_gather(tab, [f])
            cy = plsc.load_gather(tab, [f + G3])
            cz = plsc.load_gather(tab, [f + 2 * G3])
            dx, dy, dz = tx - cx, ty - cy, tz - cz
            acc_r = acc_r + ((dx * dx + dy * dy) + dz * dz)

            ux = s0 * px
            uy = s1 * py
            uz = s2 * pz
            g = (vox(ux) * G + vox(uy)) * G + vox(uz)
            qx = plsc.load_gather(tab, [g])
            qy = plsc.load_gather(tab, [g + G3])
            qz = plsc.load_gather(tab, [g + 2 * G3])
            ex, ey, ez = ux - qx, uy - qy, uz - qz
            acc_o = acc_o + ((ex * ex + ey * ey) + ez * ez)
            return acc_r, acc_o

        UN = 4
        zero = jnp.zeros((_L,), jnp.float32)
        accs = (zero,) * (2 * UN)
        for ch in range(NCHUNK):
            for cp_ in chunk_cps:
                cp_.wait()
            if ch + 1 < NCHUNK:
                next_cps = start_chunk(ch + 1)
            else:
                next_cps = []
            bufref = pbuf.at[pl.ds((ch % 2) * 3 * CH, 3 * CH)]

            def step(i, accs, bufref=bufref):
                out = []
                for k in range(UN):
                    out.extend(
                        body(bufref, i + k, accs[2 * k], accs[2 * k + 1]))
                return tuple(out)

            accs = plsc.parallel_loop(
                0, CH // _L, step=UN, carry=accs)(step)
            chunk_cps = next_cps

        acc_r = (accs[0] + accs[2]) + (accs[4] + accs[6])
        acc_o = (accs[1] + accs[3]) + (accs[5] + accs[7])
        obuf_r[...] = acc_r
        obuf_o[...] = acc_o
        pltpu.sync_copy(obuf_r, oref_ref.at[pl.ds(wid * _L, _L)])
        pltpu.sync_copy(obuf_o, orot_ref.at[pl.ds(wid * _L, _L)])

    return launch(sp_flat, cp_flat, params)


def kernel(sample_points, closest_points, bound, grid_size, planes, axes):
    B, N, _ = sample_points.shape
    G3 = closest_points.shape[1]
    G = round(G3 ** (1.0 / 3.0))
    T = planes.shape[0]

    n = planes[:, :, :3]
    d = planes[:, :, 3]
    s = jnp.sum(n * n, axis=2) + 1e-12
    inv = 2.0 / s
    w = inv[:, :, None] * n
    e = inv * d
    srot = -(axes[:, :, 1:] ** 2)

    bnd = bound[0].astype(jnp.float32)
    gsf = jnp.asarray(grid_size, jnp.float32)
    scal = jnp.stack([bnd, gsf, gsf - 1.0])

    def tb(x):
        return jnp.transpose(x, (1, 0, 2)).reshape(B * T, -1)

    params = jnp.concatenate([
        tb(n), tb(w), tb(e[:, :, None]), tb(srot),
        jnp.broadcast_to(scal, (B * T, 3)),
        jnp.zeros((B * T, 3), jnp.float32),
    ], axis=1).astype(jnp.float32)

    spT = jnp.transpose(sample_points, (2, 0, 1)).reshape(-1)
    cpT = jnp.transpose(closest_points, (2, 0, 1)).reshape(-1)

    oref, orot = _sc_symmetry_loss(spT, cpT, params.reshape(-1), N, G3, G)
    denom = jnp.float32(B * 3)
    return ((jnp.sum(oref) / denom).reshape(1),
            (jnp.sum(orot) / denom).reshape(1))
